# Initial kernel scaffold; baseline (speedup 1.0000x reference)
import jax
import jax.numpy as jnp
from jax.experimental import pallas as pl


def kernel(input_ids, token_table, pos_table):
    raise NotImplementedError("write your pallas kernel here")


# speedup vs baseline: 3.4653x; 3.4653x over previous
import jax
import jax.numpy as jnp
from jax import lax
from jax.experimental import pallas as pl
from jax.experimental.pallas import tpu as pltpu
from jax.experimental.pallas import tpu_sc as plsc

B, S, D, V = 16, 8192, 128, 100000
NC, NS = 2, 16
NW = NC * NS
S_CHUNK = S // NW
SUB = 128
NSUB = S_CHUNK // SUB


def _emb_body(ids_ref, tok_ref, pos_ref, out_ref, idx_v, rows_v, sem):
    wid = lax.axis_index("s") * NC + lax.axis_index("c")
    s0 = wid * S_CHUNK

    @pl.loop(0, B)
    def _batch(b):
        for j in range(NSUB):
            s = s0 + j * SUB
            base = b * S + s
            pltpu.sync_copy(ids_ref.at[pl.ds(base, SUB)], idx_v)
            pltpu.sync_copy(pos_ref.at[pl.ds(s, SUB)], rows_v)
            pltpu.async_copy(tok_ref.at[idx_v], rows_v, sem, add=True).wait()
            pltpu.sync_copy(rows_v, out_ref.at[pl.ds(base, SUB)])


def kernel(input_ids, token_table, pos_table):
    ids_flat = input_ids.reshape(-1).astype(jnp.int32)
    mesh = plsc.VectorSubcoreMesh(core_axis_name="c", subcore_axis_name="s")
    f = pl.kernel(
        _emb_body,
        out_type=jax.ShapeDtypeStruct((B * S, D), jnp.float32),
        mesh=mesh,
        scratch_types=[
            pltpu.VMEM((SUB,), jnp.int32),
            pltpu.VMEM((SUB, D), jnp.float32),
            pltpu.SemaphoreType.DMA,
        ],
    )
    out = f(ids_flat, token_table, pos_table)
    return out.reshape(B, S, D)


# speedup vs baseline: 5.7484x; 1.6589x over previous
import jax
import jax.numpy as jnp
from jax import lax
from jax.experimental import pallas as pl
from jax.experimental.pallas import tpu as pltpu
from jax.experimental.pallas import tpu_sc as plsc

B, S, D, V = 16, 8192, 128, 100000
NC, NS = 2, 16
NW = NC * NS
S_CHUNK = S // NW
SUB = 128
NSUB = S_CHUNK // SUB
NBUF = 4
NITER = B * NSUB // NBUF


def _emb_body(ids_ref, tok_ref, pos_ref, out_ref,
              idx, rows, sem_i, sem_p, sem_g, sem_o):
    wid = lax.axis_index("s") * NC + lax.axis_index("c")
    s0 = wid * S_CHUNK

    def srcs(bb, c):
        b = (NBUF // NSUB) * bb + c // NSUB
        s = s0 + (c % NSUB) * SUB
        return b * S + s, s

    def issue_inputs(bb, c):
        base, s = srcs(bb, c)
        pltpu.async_copy(ids_ref.at[pl.ds(base, SUB)], idx[c], sem_i[c])
        pltpu.async_copy(pos_ref.at[pl.ds(s, SUB)], rows[c], sem_p[c])

    for c in range(NBUF):
        issue_inputs(0, c)

    @pl.loop(0, NITER)
    def _iter(bb):
        gathers = []
        for c in range(NBUF):
            base, s = srcs(bb, c)
            pltpu.make_async_copy(ids_ref.at[pl.ds(base, SUB)],
                                  idx[c], sem_i[c]).wait()
            pltpu.make_async_copy(pos_ref.at[pl.ds(s, SUB)],
                                  rows[c], sem_p[c]).wait()
            gathers.append(
                pltpu.async_copy(tok_ref.at[idx[c]], rows[c], sem_g[c],
                                 add=True))
        for c in range(NBUF):
            base, _ = srcs(bb, c)
            gathers[c].wait()
            pltpu.async_copy(rows[c], out_ref.at[pl.ds(base, SUB)], sem_o[c])
        for c in range(NBUF):
            @pl.when(bb < NITER - 1)
            def _prefetch(c=c):
                base, _ = srcs(bb, c)
                pltpu.make_async_copy(rows[c], out_ref.at[pl.ds(base, SUB)],
                                      sem_o[c]).wait()
                issue_inputs(bb + 1, c)

    for c in range(NBUF):
        base, _ = srcs(NITER - 1, c)
        pltpu.make_async_copy(rows[c], out_ref.at[pl.ds(base, SUB)],
                              sem_o[c]).wait()


def kernel(input_ids, token_table, pos_table):
    ids_flat = input_ids.reshape(-1).astype(jnp.int32)
    mesh = plsc.VectorSubcoreMesh(core_axis_name="c", subcore_axis_name="s")
    f = pl.kernel(
        _emb_body,
        out_type=jax.ShapeDtypeStruct((B * S, D), jnp.float32),
        mesh=mesh,
        scratch_types=[
            [pltpu.VMEM((SUB,), jnp.int32) for _ in range(NBUF)],
            [pltpu.VMEM((SUB, D), jnp.float32) for _ in range(NBUF)],
            [pltpu.SemaphoreType.DMA for _ in range(NBUF)],
            [pltpu.SemaphoreType.DMA for _ in range(NBUF)],
            [pltpu.SemaphoreType.DMA for _ in range(NBUF)],
            [pltpu.SemaphoreType.DMA for _ in range(NBUF)],
        ],
    )
    out = f(ids_flat, token_table, pos_table)
    return out.reshape(B, S, D)
